# baseline (speedup 1.0000x reference)
import functools

import jax
import jax.numpy as jnp
from jax import lax
from jax.experimental import pallas as pl
from jax.experimental.pallas import tpu as pltpu
from jax.experimental.pallas import tpu_sc as plsc

N = 10000
E = 320000
DF = 128
NW = 32
NPT = 320
NPAD = NW * NPT
CHUNK = 8000
G = 128
NEG = float("-inf")
PI = 3.14159274101257
PI_2 = 1.57079637050629

_C = (0.99997726, -0.33262347, 0.19354346, -0.11643287, 0.05265332, -0.01172120)


def _sqrt(x):
    i = plsc.bitcast(x, jnp.int32)
    i = 0x5F3759DF - lax.shift_right_logical(i, 1)
    y = plsc.bitcast(i, jnp.float32)
    hx = 0.5 * x
    for _ in range(3):
        y = y * (1.5 - hx * y * y)
    return x * y


def _atan2_pos(y, x):
    ax = jnp.abs(x)
    mn = jnp.minimum(y, ax)
    mx = jnp.maximum(y, ax)
    a = mn / mx
    s = a * a
    p = jnp.float32(_C[5])
    for c in (_C[4], _C[3], _C[2], _C[1], _C[0]):
        p = p * s + c
    r = p * a
    r = jnp.where(y > ax, PI_2 - r, r)
    r = jnp.where(x < 0.0, PI - r, r)
    return r


def _angle(axx, ay, az, bx, by, bz):
    cx = ay * bz - az * by
    cy = az * bx - axx * bz
    cz = axx * by - ay * bx
    cn = _sqrt(cx * cx + cy * cy + cz * cz + 1e-20)
    d = axx * bx + ay * by + az * bz
    return _atan2_pos(cn, d)


def _body(dst_h, src_h, pn_h, x_h, ea_h, out_x, out_fe,
          acc_x, acc_fe, pn_own, dstb, srcb, sel_d, sel_s, sel_e,
          xr_a, xr_b, xr_c, pj_a, pj_b, pj_c, er_a, er_b, er_c,
          dm_a, dm_b, dm_c, sm_a, sm_b, sm_c, em_a, em_b, em_c,
          featb, sem_a, sem_b, sem_c):
    wid = lax.axis_index("s") * 2 + lax.axis_index("c")
    lo = wid * NPT
    ninf = jnp.full((16,), NEG, jnp.float32)
    ninfb = plsc.bitcast(jnp.full((32,), NEG, jnp.bfloat16), jnp.float32)
    zero16 = jnp.zeros((16,), jnp.int32)
    iota16 = lax.iota(jnp.int32, 16)

    pltpu.sync_copy(pn_h.at[pl.ds(lo, NPT)], pn_own)

    def init_r(r, carry):
        for c in range(4):
            acc_x[r, pl.ds(c * 16, 16)] = ninfb
        acc_fe[r, pl.ds(0, 16)] = ninf
        acc_fe[r, pl.ds(16, 16)] = ninf
        return carry
    lax.fori_loop(0, NPT, init_r, 0)

    def init_s(v, carry):
        sel_d[pl.ds(v * 16, 16)] = zero16
        sel_s[pl.ds(v * 16, 16)] = zero16
        sel_e[pl.ds(v * 16, 16)] = zero16
        return carry
    lax.fori_loop(0, CHUNK // 16, init_s, 0)

    def fire(gbase, xr, pj, er, dm, sm, em, sem):
        for v in range(G // 16):
            dm[pl.ds(v * 16, 16)] = sel_d[pl.ds(gbase + v * 16, 16)]
            sm[pl.ds(v * 16, 16)] = sel_s[pl.ds(gbase + v * 16, 16)]
            em[pl.ds(v * 16, 16)] = sel_e[pl.ds(gbase + v * 16, 16)]
        pltpu.async_copy(x_h.at[sm], xr, sem)
        pltpu.async_copy(pn_h.at[sm], pj, sem)
        pltpu.async_copy(ea_h.at[em], er, sem)

    def drain(xr, pj, er, sm, em, sem):
        pltpu.make_async_copy(x_h.at[sm], xr, sem).wait()
        pltpu.make_async_copy(pn_h.at[sm], pj, sem).wait()
        pltpu.make_async_copy(ea_h.at[em], er, sem).wait()

    NC = E // CHUNK

    def chunk_body(k, S_prev):
        base = k * CHUNK

        def run_scan():
            pltpu.sync_copy(dst_h.at[pl.ds(base, CHUNK)], dstb)
            pltpu.sync_copy(src_h.at[pl.ds(base, CHUNK)], srcb)
            UN = 5
            def scan_body(i, cnt):
                b0 = i * (16 * UN)
                c_run = cnt
                for u in range(UN):
                    off = b0 + u * 16
                    d = dstb[pl.ds(off, 16)]
                    s = srcb[pl.ds(off, 16)]
                    dl = d - lo
                    m = (dl >= 0) & (dl < NPT)
                    mi = jnp.where(m, 1, 0)
                    cs = jnp.cumsum(mi)
                    tot = cs[15]
                    pos = c_run + cs - mi
                    eid = base + off + iota16
                    plsc.store_scatter(sel_d, [pos], d, mask=m)
                    plsc.store_scatter(sel_s, [pos], s, mask=m)
                    plsc.store_scatter(sel_e, [pos], eid, mask=m)
                    c_run = c_run + tot
                return c_run
            return lax.fori_loop(0, CHUNK // (16 * UN), scan_body, 0)

        S = lax.cond(k < NC, run_scan, lambda: 0)

        def compute(gcnt, xr, pj, er, dm):
            for v in range(G // 16):
                le = v * 16 + iota16
                dv = dm[pl.ds(v * 16, 16)]
                dlv = dv - lo
                dlv = jnp.minimum(jnp.maximum(dlv, 0), NPT - 1)
                def icomp(c):
                    return plsc.load_gather(
                        pn_own, [dlv, jnp.full((16,), c, jnp.int32)])
                def jcomp(c):
                    return plsc.load_gather(
                        pj, [le, jnp.full((16,), c, jnp.int32)])
                pix, piy, piz = icomp(0), icomp(1), icomp(2)
                nix, niy, niz = icomp(3), icomp(4), icomp(5)
                pjx, pjy, pjz = jcomp(0), jcomp(1), jcomp(2)
                njx, njy, njz = jcomp(3), jcomp(4), jcomp(5)
                psx = pjx - pix
                psy = pjy - piy
                psz = pjz - piz
                f0 = _sqrt(psx * psx + psy * psy + psz * psz + 1e-20)
                f1 = _angle(nix, niy, niz, psx, psy, psz)
                f2 = _angle(njx, njy, njz, psx, psy, psz)
                f3 = _angle(nix, niy, niz, njx, njy, njz)
                fb = le * 8
                plsc.store_scatter(featb, [fb + 0], f0)
                plsc.store_scatter(featb, [fb + 1], f1)
                plsc.store_scatter(featb, [fb + 2], f2)
                plsc.store_scatter(featb, [fb + 3], f3)

            def upd(e, carry3):
                dval = dm[pl.ds(e, 16)][0]
                sl = dval - lo
                for c in range(4):
                    a = plsc.bitcast(acc_x[sl, pl.ds(c * 16, 16)],
                                     jnp.bfloat16)
                    xv = plsc.bitcast(xr[e, pl.ds(c * 16, 16)], jnp.bfloat16)
                    acc_x[sl, pl.ds(c * 16, 16)] = plsc.bitcast(
                        jnp.maximum(a, xv), jnp.float32)
                eav = er[e, :]
                a0 = acc_fe[sl, pl.ds(0, 16)]
                acc_fe[sl, pl.ds(0, 16)] = jnp.maximum(a0, eav)
                fv = featb[pl.ds(e * 8, 16)]
                a1 = acc_fe[sl, pl.ds(16, 16)]
                acc_fe[sl, pl.ds(16, 16)] = jnp.maximum(a1, fv)
                return carry3
            lax.fori_loop(0, gcnt, upd, 0)

        ngp = (S_prev + (G - 1)) // G

        @pl.when(ngp >= 1)
        def _da():
            drain(xr_a, pj_a, er_a, sm_a, em_a, sem_a)
            compute(jnp.minimum(G, S_prev), xr_a, pj_a, er_a, dm_a)

        @pl.when(ngp >= 2)
        def _db():
            drain(xr_b, pj_b, er_b, sm_b, em_b, sem_b)
            compute(jnp.minimum(G, S_prev - G), xr_b, pj_b, er_b, dm_b)

        ng = (S + (G - 1)) // G

        def ovf(q, carry2):
            fire(q * G, xr_c, pj_c, er_c, dm_c, sm_c, em_c, sem_c)
            drain(xr_c, pj_c, er_c, sm_c, em_c, sem_c)
            compute(jnp.minimum(G, S - q * G), xr_c, pj_c, er_c, dm_c)
            return carry2
        lax.fori_loop(2, jnp.maximum(ng, 2), ovf, 0)

        @pl.when(ng >= 1)
        def _fa():
            fire(0, xr_a, pj_a, er_a, dm_a, sm_a, em_a, sem_a)

        @pl.when(ng >= 2)
        def _fb():
            fire(G, xr_b, pj_b, er_b, dm_b, sm_b, em_b, sem_b)
        return S
    lax.fori_loop(0, NC + 1, chunk_body, 0)

    negb = jnp.full((32,), NEG, jnp.bfloat16)
    zerob = jnp.zeros((32,), jnp.bfloat16)

    def fin_r(r, carry):
        for c in range(4):
            v = plsc.bitcast(acc_x[r, pl.ds(c * 16, 16)], jnp.bfloat16)
            acc_x[r, pl.ds(c * 16, 16)] = plsc.bitcast(
                jnp.where(v == negb, zerob, v), jnp.float32)
        for c in range(2):
            v = acc_fe[r, pl.ds(c * 16, 16)]
            acc_fe[r, pl.ds(c * 16, 16)] = jnp.where(v == NEG, 0.0, v)
        return carry
    lax.fori_loop(0, NPT, fin_r, 0)
    pltpu.sync_copy(acc_x, out_x.at[pl.ds(lo, NPT)])
    pltpu.sync_copy(acc_fe, out_fe.at[pl.ds(lo, NPT)])


def _make_kernel():
    mesh = plsc.VectorSubcoreMesh(core_axis_name="c", subcore_axis_name="s")
    return pl.kernel(
        _body,
        mesh=mesh,
        compiler_params=pltpu.CompilerParams(
            needs_layout_passes=False, use_tc_tiling_on_sc=False),
        out_type=[
            jax.ShapeDtypeStruct((NPAD, DF // 2), jnp.float32),
            jax.ShapeDtypeStruct((NPAD, 32), jnp.float32),
        ],
        scratch_types=[
            pltpu.VMEM((NPT, DF // 2), jnp.float32),
            pltpu.VMEM((NPT, 32), jnp.float32),
            pltpu.VMEM((NPT, 8), jnp.float32),
            pltpu.VMEM((CHUNK,), jnp.int32),
            pltpu.VMEM((CHUNK,), jnp.int32),
            pltpu.VMEM((CHUNK + 16,), jnp.int32),
            pltpu.VMEM((CHUNK,), jnp.int32),
            pltpu.VMEM((CHUNK,), jnp.int32),
            pltpu.VMEM((G, 64), jnp.float32),
            pltpu.VMEM((G, 64), jnp.float32),
            pltpu.VMEM((G, 64), jnp.float32),
            pltpu.VMEM((G, 8), jnp.float32),
            pltpu.VMEM((G, 8), jnp.float32),
            pltpu.VMEM((G, 8), jnp.float32),
            pltpu.VMEM((G, 16), jnp.float32),
            pltpu.VMEM((G, 16), jnp.float32),
            pltpu.VMEM((G, 16), jnp.float32),
            pltpu.VMEM((G + 16,), jnp.int32),
            pltpu.VMEM((G + 16,), jnp.int32),
            pltpu.VMEM((G + 16,), jnp.int32),
            pltpu.VMEM((G,), jnp.int32),
            pltpu.VMEM((G,), jnp.int32),
            pltpu.VMEM((G,), jnp.int32),
            pltpu.VMEM((G,), jnp.int32),
            pltpu.VMEM((G,), jnp.int32),
            pltpu.VMEM((G,), jnp.int32),
            pltpu.VMEM((G * 8 + 16,), jnp.float32),
            pltpu.SemaphoreType.DMA,
            pltpu.SemaphoreType.DMA,
            pltpu.SemaphoreType.DMA,
        ],
    )


_sc_kernel = _make_kernel()


@jax.jit
def kernel(x, pos, normal, edge_index, local_edge_attr):
    src = edge_index[0]
    dst = edge_index[1]
    pn = jnp.concatenate(
        [pos, normal, jnp.zeros((N, 2), jnp.float32)], axis=1)
    pn = jnp.pad(pn, ((0, NPAD - N), (0, 0)))
    xb = x.astype(jnp.bfloat16).reshape(N, DF // 2, 2)
    x2 = lax.bitcast_convert_type(xb, jnp.float32)
    out_x, out_fe = _sc_kernel(dst, src, pn, x2, local_edge_attr)
    out_xf = lax.bitcast_convert_type(out_x, jnp.bfloat16)
    out_xf = out_xf.reshape(NPAD, DF).astype(jnp.float32)
    return jnp.concatenate(
        [out_xf[:N], out_fe[:N, 16:20], out_fe[:N, :16]], axis=1)


# speedup vs baseline: 1.3789x; 1.3789x over previous
import functools

import jax
import jax.numpy as jnp
from jax import lax
from jax.experimental import pallas as pl
from jax.experimental.pallas import tpu as pltpu
from jax.experimental.pallas import tpu_sc as plsc

N = 10000
E = 320000
DF = 128
NW = 32
NPT = 320
NPAD = NW * NPT
CHUNK = 6400
G = 112
NEG = float("-inf")
PI = 3.14159274101257
PI_2 = 1.57079637050629

_C = (0.99997726, -0.33262347, 0.19354346, -0.11643287, 0.05265332, -0.01172120)


def _sqrt(x):
    i = plsc.bitcast(x, jnp.int32)
    i = 0x5F3759DF - lax.shift_right_logical(i, 1)
    y = plsc.bitcast(i, jnp.float32)
    hx = 0.5 * x
    for _ in range(3):
        y = y * (1.5 - hx * y * y)
    return x * y


def _atan2_pos(y, x):
    ax = jnp.abs(x)
    mn = jnp.minimum(y, ax)
    mx = jnp.maximum(y, ax)
    a = mn / mx
    s = a * a
    p = jnp.float32(_C[5])
    for c in (_C[4], _C[3], _C[2], _C[1], _C[0]):
        p = p * s + c
    r = p * a
    r = jnp.where(y > ax, PI_2 - r, r)
    r = jnp.where(x < 0.0, PI - r, r)
    return r


def _angle(axx, ay, az, bx, by, bz):
    cx = ay * bz - az * by
    cy = az * bx - axx * bz
    cz = axx * by - ay * bx
    cn = _sqrt(cx * cx + cy * cy + cz * cz + 1e-20)
    d = axx * bx + ay * by + az * bz
    return _atan2_pos(cn, d)


def _body(dst_h, src_h, pn_h, x_h, ea_h, out_x, out_fe,
          acc_x, acc_fe, pn_own, dstb, srcb, sel_d, sel_s, sel_e,
          xr_a, xr_b, xr_c, pj_a, pj_b, pj_c, er_a, er_b, er_c,
          dm_a, dm_b, dm_c, sm_a, sm_b, sm_c, em_a, em_b, em_c,
          featb, sem_a, sem_b, sem_c):
    wid = lax.axis_index("s") * 2 + lax.axis_index("c")
    lo = wid * NPT
    ninf = jnp.full((16,), NEG, jnp.float32)
    ninfb = plsc.bitcast(jnp.full((32,), NEG, jnp.bfloat16), jnp.float32)
    zero16 = jnp.zeros((16,), jnp.int32)
    iota16 = lax.iota(jnp.int32, 16)

    pltpu.sync_copy(pn_h.at[pl.ds(lo, NPT)], pn_own)

    def init_r(r, carry):
        for c in range(4):
            acc_x[r, pl.ds(c * 16, 16)] = ninfb
        acc_fe[r, pl.ds(0, 16)] = ninf
        acc_fe[r, pl.ds(16, 16)] = ninf
        return carry
    lax.fori_loop(0, NPT, init_r, 0)

    def init_s(v, carry):
        sel_d[pl.ds(v * 16, 16)] = zero16
        sel_s[pl.ds(v * 16, 16)] = zero16
        sel_e[pl.ds(v * 16, 16)] = zero16
        return carry
    lax.fori_loop(0, CHUNK // 16, init_s, 0)

    def fire(gbase, xr, pj, er, dm, sm, em, sem):
        for v in range(G // 16):
            dm[pl.ds(v * 16, 16)] = sel_d[pl.ds(gbase + v * 16, 16)]
            sm[pl.ds(v * 16, 16)] = sel_s[pl.ds(gbase + v * 16, 16)]
            em[pl.ds(v * 16, 16)] = sel_e[pl.ds(gbase + v * 16, 16)]
        pltpu.async_copy(x_h.at[sm], xr, sem)
        pltpu.async_copy(pn_h.at[sm], pj, sem)
        pltpu.async_copy(ea_h.at[em], er, sem)

    def drain(xr, pj, er, sm, em, sem):
        pltpu.make_async_copy(x_h.at[sm], xr, sem).wait()
        pltpu.make_async_copy(pn_h.at[sm], pj, sem).wait()
        pltpu.make_async_copy(ea_h.at[em], er, sem).wait()

    NC = E // CHUNK

    def chunk_body(k, S_prev):
        base = k * CHUNK

        def run_scan():
            pltpu.sync_copy(dst_h.at[pl.ds(base, CHUNK)], dstb)
            pltpu.sync_copy(src_h.at[pl.ds(base, CHUNK)], srcb)
            UN = 8
            def scan_body(i, cnt):
                b0 = i * (16 * UN)
                c_run = cnt
                for u in range(UN):
                    off = b0 + u * 16
                    d = dstb[pl.ds(off, 16)]
                    s = srcb[pl.ds(off, 16)]
                    dl = d - lo
                    m = (dl >= 0) & (dl < NPT)
                    mi = jnp.where(m, 1, 0)
                    cs = jnp.cumsum(mi)
                    tot = cs[15]
                    pos = c_run + cs - mi
                    eid = base + off + iota16
                    plsc.store_scatter(sel_d, [pos], d, mask=m)
                    plsc.store_scatter(sel_s, [pos], s, mask=m)
                    plsc.store_scatter(sel_e, [pos], eid, mask=m)
                    c_run = c_run + tot
                return c_run
            return lax.fori_loop(0, CHUNK // (16 * UN), scan_body, 0)

        S = lax.cond(k < NC, run_scan, lambda: 0)

        def compute(gcnt, xr, pj, er, dm):
            for v in range(G // 16):
                le = v * 16 + iota16
                dv = dm[pl.ds(v * 16, 16)]
                dlv = dv - lo
                dlv = jnp.minimum(jnp.maximum(dlv, 0), NPT - 1)
                def icomp(c):
                    return plsc.load_gather(
                        pn_own, [dlv, jnp.full((16,), c, jnp.int32)])
                def jcomp(c):
                    return plsc.load_gather(
                        pj, [le, jnp.full((16,), c, jnp.int32)])
                pix, piy, piz = icomp(0), icomp(1), icomp(2)
                nix, niy, niz = icomp(3), icomp(4), icomp(5)
                pjx, pjy, pjz = jcomp(0), jcomp(1), jcomp(2)
                njx, njy, njz = jcomp(3), jcomp(4), jcomp(5)
                psx = pjx - pix
                psy = pjy - piy
                psz = pjz - piz
                f0 = _sqrt(psx * psx + psy * psy + psz * psz + 1e-20)
                f1 = _angle(nix, niy, niz, psx, psy, psz)
                f2 = _angle(njx, njy, njz, psx, psy, psz)
                f3 = _angle(nix, niy, niz, njx, njy, njz)
                fb = le * 8
                plsc.store_scatter(featb, [fb + 0], f0)
                plsc.store_scatter(featb, [fb + 1], f1)
                plsc.store_scatter(featb, [fb + 2], f2)
                plsc.store_scatter(featb, [fb + 3], f3)

            def upd(e, carry3):
                dval = dm[pl.ds(e, 16)][0]
                sl = dval - lo
                for c in range(4):
                    a = plsc.bitcast(acc_x[sl, pl.ds(c * 16, 16)],
                                     jnp.bfloat16)
                    xv = plsc.bitcast(xr[e, pl.ds(c * 16, 16)], jnp.bfloat16)
                    acc_x[sl, pl.ds(c * 16, 16)] = plsc.bitcast(
                        jnp.maximum(a, xv), jnp.float32)
                eav = er[e, :]
                a0 = acc_fe[sl, pl.ds(0, 16)]
                acc_fe[sl, pl.ds(0, 16)] = jnp.maximum(a0, eav)
                fv = featb[pl.ds(e * 8, 16)]
                a1 = acc_fe[sl, pl.ds(16, 16)]
                acc_fe[sl, pl.ds(16, 16)] = jnp.maximum(a1, fv)
                return carry3
            lax.fori_loop(0, gcnt, upd, 0)

        ngp = (S_prev + (G - 1)) // G

        @pl.when(ngp >= 1)
        def _da():
            drain(xr_a, pj_a, er_a, sm_a, em_a, sem_a)
            compute(jnp.minimum(G, S_prev), xr_a, pj_a, er_a, dm_a)

        @pl.when(ngp >= 2)
        def _db():
            drain(xr_b, pj_b, er_b, sm_b, em_b, sem_b)
            compute(jnp.minimum(G, S_prev - G), xr_b, pj_b, er_b, dm_b)

        ng = (S + (G - 1)) // G

        def ovf(q, carry2):
            fire(q * G, xr_c, pj_c, er_c, dm_c, sm_c, em_c, sem_c)
            drain(xr_c, pj_c, er_c, sm_c, em_c, sem_c)
            compute(jnp.minimum(G, S - q * G), xr_c, pj_c, er_c, dm_c)
            return carry2
        lax.fori_loop(2, jnp.maximum(ng, 2), ovf, 0)

        @pl.when(ng >= 1)
        def _fa():
            fire(0, xr_a, pj_a, er_a, dm_a, sm_a, em_a, sem_a)

        @pl.when(ng >= 2)
        def _fb():
            fire(G, xr_b, pj_b, er_b, dm_b, sm_b, em_b, sem_b)
        return S
    lax.fori_loop(0, NC + 1, chunk_body, 0)

    negb = jnp.full((32,), NEG, jnp.bfloat16)
    zerob = jnp.zeros((32,), jnp.bfloat16)

    def fin_r(r, carry):
        for c in range(4):
            v = plsc.bitcast(acc_x[r, pl.ds(c * 16, 16)], jnp.bfloat16)
            acc_x[r, pl.ds(c * 16, 16)] = plsc.bitcast(
                jnp.where(v == negb, zerob, v), jnp.float32)
        for c in range(2):
            v = acc_fe[r, pl.ds(c * 16, 16)]
            acc_fe[r, pl.ds(c * 16, 16)] = jnp.where(v == NEG, 0.0, v)
        return carry
    lax.fori_loop(0, NPT, fin_r, 0)
    pltpu.sync_copy(acc_x, out_x.at[pl.ds(lo, NPT)])
    pltpu.sync_copy(acc_fe, out_fe.at[pl.ds(lo, NPT)])


def _make_kernel():
    mesh = plsc.VectorSubcoreMesh(core_axis_name="c", subcore_axis_name="s")
    return pl.kernel(
        _body,
        mesh=mesh,
        compiler_params=pltpu.CompilerParams(
            needs_layout_passes=False, use_tc_tiling_on_sc=False),
        out_type=[
            jax.ShapeDtypeStruct((NPAD, DF // 2), jnp.float32),
            jax.ShapeDtypeStruct((NPAD, 32), jnp.float32),
        ],
        scratch_types=[
            pltpu.VMEM((NPT, DF // 2), jnp.float32),
            pltpu.VMEM((NPT, 32), jnp.float32),
            pltpu.VMEM((NPT, 8), jnp.float32),
            pltpu.VMEM((CHUNK,), jnp.int32),
            pltpu.VMEM((CHUNK,), jnp.int32),
            pltpu.VMEM((CHUNK + 16,), jnp.int32),
            pltpu.VMEM((CHUNK,), jnp.int32),
            pltpu.VMEM((CHUNK,), jnp.int32),
            pltpu.VMEM((G, 64), jnp.float32),
            pltpu.VMEM((G, 64), jnp.float32),
            pltpu.VMEM((G, 64), jnp.float32),
            pltpu.VMEM((G, 8), jnp.float32),
            pltpu.VMEM((G, 8), jnp.float32),
            pltpu.VMEM((G, 8), jnp.float32),
            pltpu.VMEM((G, 16), jnp.float32),
            pltpu.VMEM((G, 16), jnp.float32),
            pltpu.VMEM((G, 16), jnp.float32),
            pltpu.VMEM((G + 16,), jnp.int32),
            pltpu.VMEM((G + 16,), jnp.int32),
            pltpu.VMEM((G + 16,), jnp.int32),
            pltpu.VMEM((G,), jnp.int32),
            pltpu.VMEM((G,), jnp.int32),
            pltpu.VMEM((G,), jnp.int32),
            pltpu.VMEM((G,), jnp.int32),
            pltpu.VMEM((G,), jnp.int32),
            pltpu.VMEM((G,), jnp.int32),
            pltpu.VMEM((G * 8 + 16,), jnp.float32),
            pltpu.SemaphoreType.DMA,
            pltpu.SemaphoreType.DMA,
            pltpu.SemaphoreType.DMA,
        ],
    )


_sc_kernel = _make_kernel()


@jax.jit
def kernel(x, pos, normal, edge_index, local_edge_attr):
    src = edge_index[0]
    dst = edge_index[1]
    pn = jnp.concatenate(
        [pos, normal, jnp.zeros((N, 2), jnp.float32)], axis=1)
    pn = jnp.pad(pn, ((0, NPAD - N), (0, 0)))
    xb = x.astype(jnp.bfloat16).reshape(N, DF // 2, 2)
    x2 = lax.bitcast_convert_type(xb, jnp.float32)
    out_x, out_fe = _sc_kernel(dst, src, pn, x2, local_edge_attr)
    out_xf = lax.bitcast_convert_type(out_x, jnp.bfloat16)
    out_xf = out_xf.reshape(NPAD, DF).astype(jnp.float32)
    return jnp.concatenate(
        [out_xf[:N], out_fe[:N, 16:20], out_fe[:N, :16]], axis=1)
